# baseline (speedup 1.0000x reference)
import functools

import jax
import jax.numpy as jnp
from jax import lax
from jax.experimental import pallas as pl
from jax.experimental.pallas import tpu as pltpu
from jax.experimental.pallas import tpu_sc as plsc

N = 10000
E = 320000
D = 128

NC = 2
NS = 16
NW = NC * NS

EDGES_PER_TILE = E // NW
CHUNK = 80
NCHUNK = EDGES_PER_TILE // CHUNK
EPT_PAD = NCHUNK * CHUNK
SLAB = 80
NSLAB = N // SLAB
SLAB_ITERS = -(-NSLAB // NS)
N_ACC = N

_sc_mesh = plsc.VectorSubcoreMesh(core_axis_name="c", subcore_axis_name="s")


@functools.partial(
    pl.kernel,
    mesh=_sc_mesh,
    out_type=jax.ShapeDtypeStruct((NC * N, D), jnp.float32),
    scratch_types=[
        pltpu.VMEM((EPT_PAD,), jnp.int32),
        pltpu.VMEM((NCHUNK, CHUNK), jnp.int32),
        pltpu.VMEM((CHUNK, D), jnp.float32),
        pltpu.VMEM((CHUNK, D), jnp.float32),
        pltpu.VMEM_SHARED((N_ACC, D), jnp.float32),
        pltpu.SemaphoreType.DMA,
        pltpu.SemaphoreType.DMA,
        pltpu.SemaphoreType.DMA,
        pltpu.SemaphoreType.DMA,
    ],
)
def _sc_segment_sum(h_hbm, src_hbm, dst_hbm, parts_hbm,
                    src_v, dst_v, rows0_v, rows1_v, agg_sh,
                    sem0, sem1, sem2, sem3):
    cid = lax.axis_index("c")
    sid = lax.axis_index("s")
    wid = cid * NS + sid
    rows_b = (rows0_v, rows1_v)
    sem_b = (sem0, sem1)

    pltpu.async_copy(src_hbm.at[wid], src_v, sem2)
    pltpu.async_copy(dst_hbm.at[wid], dst_v, sem3)

    stage_v = rows1_v.at[pl.ds(0, SLAB)]

    def zbody(t, carry):
        r = t // (D // 16)
        c = (t % (D // 16)) * 16
        stage_v[r, pl.ds(c, 16)] = jnp.zeros((16,), jnp.float32)
        return carry

    lax.fori_loop(0, SLAB * (D // 16), zbody, 0)

    def start(t, b):
        idx = src_v.at[pl.ds(t * CHUNK, CHUNK)]
        pltpu.async_copy(h_hbm.at[idx], rows_b[b], sem_b[b])

    def finish(t, b):
        idx = src_v.at[pl.ds(t * CHUNK, CHUNK)]
        pltpu.make_async_copy(h_hbm.at[idx], rows_b[b], sem_b[b]).wait()
        pltpu.sync_copy(rows_b[b], agg_sh.at[dst_v.at[t]], add=True)

    pltpu.make_async_copy(src_hbm.at[wid], src_v, sem2).wait()
    start(0, 0)

    for s in range(SLAB_ITERS):
        slab = s * NS + sid

        @pl.when(slab < NSLAB)
        def _():
            pltpu.async_copy(stage_v, agg_sh.at[pl.ds(slab * SLAB, SLAB)], sem1)

    for s in range(SLAB_ITERS):
        slab = s * NS + sid

        @pl.when(slab < NSLAB)
        def _():
            pltpu.make_async_copy(
                stage_v, agg_sh.at[pl.ds(slab * SLAB, SLAB)], sem1).wait()

    pltpu.make_async_copy(dst_hbm.at[wid], dst_v, sem3).wait()
    plsc.subcore_barrier()


    def ebody(g, carry):
        t0 = g * 2
        start(t0 + 1, 1)
        finish(t0, 0)
        start(t0 + 2, 0)
        finish(t0 + 1, 1)
        return carry

    if NCHUNK % 2 == 1:
        lax.fori_loop(0, (NCHUNK - 1) // 2, ebody, 0)
        finish(NCHUNK - 1, 0)
    else:
        lax.fori_loop(0, (NCHUNK - 2) // 2, ebody, 0)
        start(NCHUNK - 1, 1)
        finish(NCHUNK - 2, 0)
        finish(NCHUNK - 1, 1)
    plsc.subcore_barrier()

    sem_in = (sem0, sem1)
    sem_out = (sem2, sem3)

    stage_b = (rows0_v.at[pl.ds(0, SLAB)], rows1_v.at[pl.ds(0, SLAB)])

    def ostart(s):
        slab = s * NS + sid

        @pl.when(slab < NSLAB)
        def _():
            pltpu.async_copy(agg_sh.at[pl.ds(slab * SLAB, SLAB)],
                             stage_b[s % 2], sem_in[s % 2])

    def ofinish(s):
        slab = s * NS + sid

        @pl.when(slab < NSLAB)
        def _():
            row0 = slab * SLAB
            pltpu.make_async_copy(agg_sh.at[pl.ds(row0, SLAB)],
                                  stage_b[s % 2], sem_in[s % 2]).wait()
            pltpu.async_copy(stage_b[s % 2],
                             parts_hbm.at[pl.ds(cid * N + row0, SLAB)],
                             sem_out[s % 2])

    def odrain(s):
        slab = s * NS + sid

        @pl.when(slab < NSLAB)
        def _():
            pltpu.make_async_copy(
                stage_b[s % 2],
                parts_hbm.at[pl.ds(cid * N + slab * SLAB, SLAB)],
                sem_out[s % 2]).wait()

    ostart(0)
    for s in range(SLAB_ITERS):
        if s >= 1:
            odrain(s - 1)
        if s + 1 < SLAB_ITERS:
            ostart(s + 1)
        ofinish(s)
    odrain(SLAB_ITERS - 1)


_TC_BLK = 1000
_TC_GRID = N // _TC_BLK


def _prescale_body(feat_ref, onorm_ref, h_ref):
    h_ref[...] = feat_ref[...] / onorm_ref[...]


def _final_body(p0_ref, p1_ref, inorm_ref, wt_ref, b_ref, o_ref):
    x = (p0_ref[...] + p1_ref[...]) / inorm_ref[...]
    o_ref[...] = (
        jnp.dot(x, wt_ref[...], preferred_element_type=jnp.float32) + b_ref[...]
    )


def kernel(feat, in_norm, out_norm, edge_index, W, b):
    h = pl.pallas_call(
        _prescale_body,
        grid=(_TC_GRID,),
        in_specs=[
            pl.BlockSpec((_TC_BLK, D), lambda i: (i, 0)),
            pl.BlockSpec((_TC_BLK, 1), lambda i: (i, 0)),
        ],
        out_specs=pl.BlockSpec((_TC_BLK, D), lambda i: (i, 0)),
        out_shape=jax.ShapeDtypeStruct((N, D), jnp.float32),
    )(feat, out_norm[:, None])

    src2 = edge_index[0].reshape(NW, EDGES_PER_TILE)
    dst3 = edge_index[1].reshape(NW, NCHUNK, CHUNK)
    parts = _sc_segment_sum(h, src2, dst3)

    out = pl.pallas_call(
        _final_body,
        grid=(_TC_GRID,),
        in_specs=[
            pl.BlockSpec((_TC_BLK, D), lambda i: (i, 0)),
            pl.BlockSpec((_TC_BLK, D), lambda i: (N // _TC_BLK + i, 0)),
            pl.BlockSpec((_TC_BLK, 1), lambda i: (i, 0)),
            pl.BlockSpec((D, D), lambda i: (0, 0)),
            pl.BlockSpec((1, D), lambda i: (0, 0)),
        ],
        out_specs=pl.BlockSpec((_TC_BLK, D), lambda i: (i, 0)),
        out_shape=jax.ShapeDtypeStruct((N, D), jnp.float32),
    )(parts, parts, in_norm[:, None], W.T, b[None, :])
    return out


# speedup vs baseline: 1.0291x; 1.0291x over previous
import functools

import jax
import jax.numpy as jnp
from jax import lax
from jax.experimental import pallas as pl
from jax.experimental.pallas import tpu as pltpu
from jax.experimental.pallas import tpu_sc as plsc

N = 10000
E = 320000
D = 128

NC = 2
NS = 16
NW = NC * NS

EDGES_PER_TILE = E // NW
CHUNK = 80
NCHUNK = EDGES_PER_TILE // CHUNK
EPT_PAD = NCHUNK * CHUNK
SLAB = 80
NSLAB = N // SLAB
SLAB_ITERS = -(-NSLAB // NS)
N_ACC = N

_sc_mesh = plsc.VectorSubcoreMesh(core_axis_name="c", subcore_axis_name="s")


@functools.partial(
    pl.kernel,
    mesh=_sc_mesh,
    out_type=jax.ShapeDtypeStruct((NC * N, D), jnp.float32),
    scratch_types=[
        pltpu.VMEM((EPT_PAD,), jnp.int32),
        pltpu.VMEM((NCHUNK, CHUNK), jnp.int32),
        pltpu.VMEM((CHUNK, D), jnp.float32),
        pltpu.VMEM((CHUNK, D), jnp.float32),
        pltpu.VMEM_SHARED((N_ACC, D), jnp.float32),
        pltpu.SemaphoreType.DMA,
        pltpu.SemaphoreType.DMA,
        pltpu.SemaphoreType.DMA,
        pltpu.SemaphoreType.DMA,
    ],
)
def _sc_segment_sum(h_hbm, src_hbm, dst_hbm, parts_hbm,
                    src_v, dst_v, rows0_v, rows1_v, agg_sh,
                    sem0, sem1, sem2, sem3):
    cid = lax.axis_index("c")
    sid = lax.axis_index("s")
    wid = cid * NS + sid
    rows_b = (rows0_v, rows1_v)
    sem_b = (sem0, sem1)

    pltpu.async_copy(src_hbm.at[wid], src_v, sem2)
    pltpu.async_copy(dst_hbm.at[wid], dst_v, sem3)

    stage_v = rows1_v.at[pl.ds(0, SLAB)]

    def zbody(t, carry):
        r = t // (D // 16)
        c = (t % (D // 16)) * 16
        stage_v[r, pl.ds(c, 16)] = jnp.zeros((16,), jnp.float32)
        return carry

    lax.fori_loop(0, SLAB * (D // 16), zbody, 0)

    def start(t, b):
        idx = src_v.at[pl.ds(t * CHUNK, CHUNK)]
        pltpu.async_copy(h_hbm.at[idx], rows_b[b], sem_b[b])

    def finish(t, b):
        idx = src_v.at[pl.ds(t * CHUNK, CHUNK)]
        pltpu.make_async_copy(h_hbm.at[idx], rows_b[b], sem_b[b]).wait()
        pltpu.sync_copy(rows_b[b], agg_sh.at[dst_v.at[t]], add=True)

    pltpu.make_async_copy(src_hbm.at[wid], src_v, sem2).wait()
    start(0, 0)

    for s in range(SLAB_ITERS):
        slab = s * NS + sid

        @pl.when(slab < NSLAB)
        def _():
            pltpu.async_copy(stage_v, agg_sh.at[pl.ds(slab * SLAB, SLAB)], sem1)

    for s in range(SLAB_ITERS):
        slab = s * NS + sid

        @pl.when(slab < NSLAB)
        def _():
            pltpu.make_async_copy(
                stage_v, agg_sh.at[pl.ds(slab * SLAB, SLAB)], sem1).wait()

    pltpu.make_async_copy(dst_hbm.at[wid], dst_v, sem3).wait()
    plsc.subcore_barrier()


    def ebody(g, carry):
        t0 = g * 2
        start(t0 + 1, 1)
        finish(t0, 0)
        start(t0 + 2, 0)
        finish(t0 + 1, 1)
        return carry

    if NCHUNK % 2 == 1:
        lax.fori_loop(0, (NCHUNK - 1) // 2, ebody, 0)
        finish(NCHUNK - 1, 0)
    else:
        lax.fori_loop(0, (NCHUNK - 2) // 2, ebody, 0)
        start(NCHUNK - 1, 1)
        finish(NCHUNK - 2, 0)
        finish(NCHUNK - 1, 1)
    plsc.subcore_barrier()

    sem_in = (sem0, sem1)
    sem_out = (sem2, sem3)

    stage_b = (rows0_v.at[pl.ds(0, SLAB)], rows1_v.at[pl.ds(0, SLAB)])

    def ostart(s):
        slab = s * NS + sid

        @pl.when(slab < NSLAB)
        def _():
            pltpu.async_copy(agg_sh.at[pl.ds(slab * SLAB, SLAB)],
                             stage_b[s % 2], sem_in[s % 2])

    def ofinish(s):
        slab = s * NS + sid

        @pl.when(slab < NSLAB)
        def _():
            row0 = slab * SLAB
            pltpu.make_async_copy(agg_sh.at[pl.ds(row0, SLAB)],
                                  stage_b[s % 2], sem_in[s % 2]).wait()
            pltpu.async_copy(stage_b[s % 2],
                             parts_hbm.at[pl.ds(cid * N + row0, SLAB)],
                             sem_out[s % 2])

    def odrain(s):
        slab = s * NS + sid

        @pl.when(slab < NSLAB)
        def _():
            pltpu.make_async_copy(
                stage_b[s % 2],
                parts_hbm.at[pl.ds(cid * N + slab * SLAB, SLAB)],
                sem_out[s % 2]).wait()

    ostart(0)
    for s in range(SLAB_ITERS):
        if s >= 1:
            odrain(s - 1)
        if s + 1 < SLAB_ITERS:
            ostart(s + 1)
        ofinish(s)
    odrain(SLAB_ITERS - 1)


_TC_BLK = 2000
_TC_GRID = N // _TC_BLK


def _prescale_body(feat_ref, onorm_ref, h_ref):
    h_ref[...] = feat_ref[...] / onorm_ref[...]


def _final_body(p0_ref, p1_ref, inorm_ref, wt_ref, b_ref, o_ref):
    x = (p0_ref[...] + p1_ref[...]) / inorm_ref[...]
    o_ref[...] = (
        jnp.dot(x, wt_ref[...], preferred_element_type=jnp.float32) + b_ref[...]
    )


def kernel(feat, in_norm, out_norm, edge_index, W, b):
    h = pl.pallas_call(
        _prescale_body,
        grid=(_TC_GRID,),
        in_specs=[
            pl.BlockSpec((_TC_BLK, D), lambda i: (i, 0)),
            pl.BlockSpec((_TC_BLK, 1), lambda i: (i, 0)),
        ],
        out_specs=pl.BlockSpec((_TC_BLK, D), lambda i: (i, 0)),
        out_shape=jax.ShapeDtypeStruct((N, D), jnp.float32),
    )(feat, out_norm[:, None])

    src2 = edge_index[0].reshape(NW, EDGES_PER_TILE)
    dst3 = edge_index[1].reshape(NW, NCHUNK, CHUNK)
    parts = _sc_segment_sum(h, src2, dst3)

    out = pl.pallas_call(
        _final_body,
        grid=(_TC_GRID,),
        in_specs=[
            pl.BlockSpec((_TC_BLK, D), lambda i: (i, 0)),
            pl.BlockSpec((_TC_BLK, D), lambda i: (N // _TC_BLK + i, 0)),
            pl.BlockSpec((_TC_BLK, 1), lambda i: (i, 0)),
            pl.BlockSpec((D, D), lambda i: (0, 0)),
            pl.BlockSpec((1, D), lambda i: (0, 0)),
        ],
        out_specs=pl.BlockSpec((_TC_BLK, D), lambda i: (i, 0)),
        out_shape=jax.ShapeDtypeStruct((N, D), jnp.float32),
    )(parts, parts, in_norm[:, None], W.T, b[None, :])
    return out
